# baseline (speedup 1.0000x reference)
import math

import jax
import jax.numpy as jnp
from jax import lax
from jax.experimental import pallas as pl
from jax.experimental.pallas import tpu as pltpu
from jax.experimental.pallas import tpu_sc as plsc

EMB = 32
SCALE = math.sqrt(EMB)
CN = 128


def _make_kernel(B, V):
    info = plsc.get_sparse_core_info()
    NC = info.num_cores
    NW = NC * info.num_subcores
    RW = B // NW
    NCH = RW // CN
    assert NCH % 2 == 0

    mesh = plsc.VectorSubcoreMesh(core_axis_name="c", subcore_axis_name="s")

    @pl.kernel(
        mesh=mesh,
        out_type=jax.ShapeDtypeStruct((B, EMB), jnp.float32),
        scratch_types=[
            pltpu.VMEM((2, CN), jnp.int32),
            pltpu.VMEM((2, CN), jnp.int32),
            pltpu.VMEM((2, CN, 128), jnp.float32),
            pltpu.VMEM((2, CN, EMB), jnp.float32),
            pltpu.SemaphoreType.DMA,
            pltpu.SemaphoreType.DMA,
            pltpu.SemaphoreType.DMA,
            pltpu.SemaphoreType.DMA,
            pltpu.SemaphoreType.DMA,
            pltpu.SemaphoreType.DMA,
        ],
        compiler_params=pltpu.CompilerParams(needs_layout_passes=False),
    )
    def k(tok_hbm, tbl_hbm, out_hbm, idx_v, gid_v, grp_v, row_v,
          si0, si1, sg0, sg1, so0, so1):
        wid = lax.axis_index("s") * NC + lax.axis_index("c")
        base = wid * RW
        trow0 = wid * NCH
        iota16 = lax.iota(jnp.int32, 16)
        semi = (si0, si1)
        semg = (sg0, sg1)
        semo = (so0, so1)

        def fire_idx(c, b):
            pltpu.async_copy(tok_hbm.at[trow0 + c], idx_v.at[b], semi[b])

        def wait_idx(b):
            pltpu.make_async_copy(
                tok_hbm.at[trow0], idx_v.at[b], semi[b]).wait()

        def prep_gid(b):
            for j in range(CN // 16):
                sl = (b, pl.ds(j * 16, 16))
                gid_v[sl] = lax.shift_right_logical(idx_v[sl], 2)

        def fire_gather(b):
            pltpu.async_copy(tbl_hbm.at[gid_v.at[b]], grp_v.at[b], semg[b])

        def wait_gather(b):
            pltpu.make_async_copy(
                tbl_hbm.at[pl.ds(0, CN)], grp_v.at[b], semg[b]).wait()

        def out_slice(ci):
            return out_hbm.at[pl.ds(pl.multiple_of(base + ci * CN, 8), CN)]

        def extract(b):
            for i0 in range(0, CN, 16):
                tv = idx_v[b, pl.ds(i0, 16)]
                cb16 = lax.shift_left(
                    lax.bitwise_and(tv, jnp.full((16,), 3, jnp.int32)), 5)
                i16 = iota16 + i0

                def col_body(c4, carry):
                    for cc in range(8):
                        c16 = jnp.full((16,), 1, jnp.int32) * (c4 * 8 + cc)
                        vals = plsc.load_gather(
                            grp_v.at[b], [i16, cb16 + c16])
                        plsc.store_scatter(row_v.at[b], [i16, c16],
                                           vals * SCALE)
                    return carry
                lax.fori_loop(0, EMB // 8, col_body, 0)

        pltpu.sync_copy(tok_hbm.at[trow0], idx_v.at[0])
        prep_gid(0)
        fire_gather(0)

        def outer(c2, carry):
            for b in range(2):
                ci = c2 * 2 + b
                nb = 1 - b

                @pl.when(ci + 1 < NCH)
                def _fire_next_idx():
                    fire_idx(ci + 1, nb)

                wait_gather(b)

                @pl.when(ci >= 2)
                def _drain_out():
                    pltpu.make_async_copy(
                        row_v.at[b], out_slice(ci - 2), semo[b]).wait()

                @pl.when(ci + 1 < NCH)
                def _fire_next_gather():
                    wait_idx(nb)
                    prep_gid(nb)
                    fire_gather(nb)

                extract(b)
                pltpu.async_copy(row_v.at[b], out_slice(ci), semo[b])
            return carry

        lax.fori_loop(0, NCH // 2, outer, 0)
        pltpu.make_async_copy(
            row_v.at[0], out_slice(NCH - 2), semo[0]).wait()
        pltpu.make_async_copy(
            row_v.at[1], out_slice(NCH - 1), semo[1]).wait()

    return k


def kernel(tokens, embedding):
    B = tokens.shape[0] * tokens.shape[1]
    V = embedding.shape[0]
    tok2d = tokens.reshape(B // CN, CN).astype(jnp.int32)
    tbl128 = embedding.reshape(V * EMB // 128, 128)
    out = _make_kernel(B, V)(tok2d, tbl128)
    return out.reshape(tokens.shape[0], tokens.shape[1], EMB)


# speedup vs baseline: 1.2280x; 1.2280x over previous
import math

import jax
import jax.numpy as jnp
from jax import lax
from jax.experimental import pallas as pl
from jax.experimental.pallas import tpu as pltpu
from jax.experimental.pallas import tpu_sc as plsc

EMB = 32
SCALE = math.sqrt(EMB)
CN = 128


def _make_kernel(B, V):
    info = plsc.get_sparse_core_info()
    NC = info.num_cores
    NW = NC * info.num_subcores
    RW = B // NW
    NCH = RW // CN
    assert NCH % 2 == 0

    mesh = plsc.VectorSubcoreMesh(core_axis_name="c", subcore_axis_name="s")

    @pl.kernel(
        mesh=mesh,
        out_type=jax.ShapeDtypeStruct((B, EMB), jnp.float32),
        scratch_types=[
            pltpu.VMEM((2, CN), jnp.int32),
            pltpu.VMEM((2, CN), jnp.int32),
            pltpu.VMEM((2, CN, 128), jnp.float32),
            pltpu.VMEM((2, CN, EMB), jnp.float32),
            pltpu.SemaphoreType.DMA,
            pltpu.SemaphoreType.DMA,
            pltpu.SemaphoreType.DMA,
            pltpu.SemaphoreType.DMA,
            pltpu.SemaphoreType.DMA,
            pltpu.SemaphoreType.DMA,
        ],
        compiler_params=pltpu.CompilerParams(needs_layout_passes=False),
    )
    def k(tok_hbm, tbl_hbm, out_hbm, idx_v, gid_v, grp_v, row_v,
          si0, si1, sg0, sg1, so0, so1):
        wid = lax.axis_index("s") * NC + lax.axis_index("c")
        base = wid * RW
        trow0 = wid * NCH
        iota16 = lax.iota(jnp.int32, 16)
        semi = (si0, si1)
        semg = (sg0, sg1)
        semo = (so0, so1)

        def fire_idx(c, b):
            pltpu.async_copy(tok_hbm.at[trow0 + c], idx_v.at[b], semi[b])

        def wait_idx(b):
            pltpu.make_async_copy(
                tok_hbm.at[trow0], idx_v.at[b], semi[b]).wait()

        def prep_gid(b):
            for j in range(CN // 16):
                sl = (b, pl.ds(j * 16, 16))
                gid_v[sl] = lax.shift_right_logical(idx_v[sl], 2)

        def fire_gather(b):
            pltpu.async_copy(tbl_hbm.at[gid_v.at[b]], grp_v.at[b], semg[b])

        def wait_gather(b):
            pltpu.make_async_copy(
                tbl_hbm.at[pl.ds(0, CN)], grp_v.at[b], semg[b]).wait()

        def out_slice(ci):
            return out_hbm.at[pl.ds(pl.multiple_of(base + ci * CN, 8), CN)]

        def extract(b):
            for i0 in range(0, CN, 16):
                tv = idx_v[b, pl.ds(i0, 16)]
                cb16 = lax.shift_left(
                    lax.bitwise_and(tv, jnp.full((16,), 3, jnp.int32)), 5)
                i16 = iota16 + i0
                for cb in range(0, EMB, 16):
                    vals = []
                    for c in range(cb, cb + 16):
                        c16 = jnp.full((16,), c, jnp.int32)
                        vals.append(plsc.load_gather(
                            grp_v.at[b], [i16, cb16 + c16]) * SCALE)
                    for j, c in enumerate(range(cb, cb + 16)):
                        c16 = jnp.full((16,), c, jnp.int32)
                        plsc.store_scatter(row_v.at[b], [i16, c16], vals[j])

        pltpu.sync_copy(tok_hbm.at[trow0], idx_v.at[0])
        prep_gid(0)
        fire_gather(0)

        def outer(c2, carry):
            for b in range(2):
                ci = c2 * 2 + b
                nb = 1 - b

                @pl.when(ci + 1 < NCH)
                def _fire_next_idx():
                    fire_idx(ci + 1, nb)

                wait_gather(b)

                @pl.when(ci >= 2)
                def _drain_out():
                    pltpu.make_async_copy(
                        row_v.at[b], out_slice(ci - 2), semo[b]).wait()

                @pl.when(ci + 1 < NCH)
                def _fire_next_gather():
                    wait_idx(nb)
                    prep_gid(nb)
                    fire_gather(nb)

                extract(b)
                pltpu.async_copy(row_v.at[b], out_slice(ci), semo[b])
            return carry

        lax.fori_loop(0, NCH // 2, outer, 0)
        pltpu.make_async_copy(
            row_v.at[0], out_slice(NCH - 2), semo[0]).wait()
        pltpu.make_async_copy(
            row_v.at[1], out_slice(NCH - 1), semo[1]).wait()

    return k


def kernel(tokens, embedding):
    B = tokens.shape[0] * tokens.shape[1]
    V = embedding.shape[0]
    tok2d = tokens.reshape(B // CN, CN).astype(jnp.int32)
    tbl128 = embedding.reshape(V * EMB // 128, 128)
    out = _make_kernel(B, V)(tok2d, tbl128)
    return out.reshape(tokens.shape[0], tokens.shape[1], EMB)


# speedup vs baseline: 1.7371x; 1.4146x over previous
import math

import jax
import jax.numpy as jnp
from jax import lax
from jax.experimental import pallas as pl
from jax.experimental.pallas import tpu as pltpu
from jax.experimental.pallas import tpu_sc as plsc

EMB = 32
SCALE = math.sqrt(EMB)


CT = 320


def _make_compactor(V):
    info = plsc.get_sparse_core_info()
    NC = info.num_cores
    NW = NC * info.num_subcores
    NCHT = V // CT
    base_n = NCHT // NW
    extra = NCHT - base_n * NW

    mesh = plsc.VectorSubcoreMesh(core_axis_name="c", subcore_axis_name="s")

    @pl.kernel(
        mesh=mesh,
        out_type=jax.ShapeDtypeStruct((V * EMB // 128, 128), jnp.float32),
        scratch_types=[
            pltpu.VMEM((2, CT, EMB), jnp.float32),
            pltpu.VMEM((2, CT * EMB // 128, 128), jnp.float32),
            pltpu.SemaphoreType.DMA,
            pltpu.SemaphoreType.DMA,
            pltpu.SemaphoreType.DMA,
            pltpu.SemaphoreType.DMA,
        ],
        compiler_params=pltpu.CompilerParams(needs_layout_passes=False),
    )
    def k(tbl_hbm, out_hbm, vin, vout, sr0, sr1, sw0, sw1):
        wid = lax.axis_index("s") * NC + lax.axis_index("c")
        nw = base_n + jnp.where(wid < extra, 1, 0)
        semr = (sr0, sr1)
        semw = (sw0, sw1)

        def src(kk):
            r0 = pl.multiple_of((wid + kk * NW) * CT, 8)
            return tbl_hbm.at[pl.ds(r0, CT)]

        def dst(kk):
            r0 = pl.multiple_of((wid + kk * NW) * (CT * EMB // 128), 8)
            return out_hbm.at[pl.ds(r0, CT * EMB // 128)]

        def repack(b):
            def body(i, carry):
                r = i * 16
                vals = []
                for u in range(16):
                    for h in range(EMB // 16):
                        vals.append(vin[b, r + u, pl.ds(h * 16, 16)] * SCALE)
                for u in range(16):
                    for h in range(EMB // 16):
                        d = (b, r // 4 + u // 4,
                             pl.ds((u % 4) * EMB + h * 16, 16))
                        vout[d] = vals[u * (EMB // 16) + h]
                return carry
            lax.fori_loop(0, CT // 16, body, 0)

        @pl.when(nw > 0)
        def _prologue():
            pltpu.async_copy(src(0), vin.at[0], semr[0])

        def outer(t2, carry):
            for half in range(2):
                kk = t2 * 2 + half
                b = half
                nb = 1 - half

                @pl.when(kk + 1 < nw)
                def _fire_read():
                    pltpu.async_copy(src(kk + 1), vin.at[nb], semr[nb])

                @pl.when(kk < nw)
                def _proc():
                    pltpu.make_async_copy(src(kk), vin.at[b], semr[b]).wait()

                    @pl.when(kk >= 2)
                    def _drain_w():
                        pltpu.make_async_copy(
                            vout.at[b], dst(kk - 2), semw[b]).wait()
                    repack(b)
                    pltpu.async_copy(vout.at[b], dst(kk), semw[b])
            return carry

        lax.fori_loop(0, (base_n + 2) // 2, outer, 0)

        @pl.when(nw >= 2)
        def _drain0():
            pltpu.make_async_copy(vout.at[0], dst(0), semw[0]).wait()

        @pl.when(nw >= 1)
        def _drain1():
            pltpu.make_async_copy(vout.at[1], dst(0), semw[1]).wait()

    return k


K = 128
CH = 5
C = K * CH


def _make_gather(B, V):
    info = plsc.get_sparse_core_info()
    NC = info.num_cores
    NW = NC * info.num_subcores
    RW = B // NW
    NCH = RW // C
    assert NCH % 2 == 0 and NCH * C == RW

    mesh = plsc.VectorSubcoreMesh(core_axis_name="c", subcore_axis_name="s")

    @pl.kernel(
        mesh=mesh,
        out_type=jax.ShapeDtypeStruct((B * EMB // 128, 128), jnp.float32),
        scratch_types=[
            pltpu.VMEM((RW // K, K), jnp.int32),
            pltpu.VMEM((2, C, EMB), jnp.float32),
            pltpu.VMEM((2, C * EMB // 128, 128), jnp.float32),
            pltpu.SemaphoreType.DMA,
            pltpu.SemaphoreType.DMA,
            pltpu.SemaphoreType.DMA,
            pltpu.SemaphoreType.DMA,
        ],
        compiler_params=pltpu.CompilerParams(use_tc_tiling_on_sc=False),
    )
    def k(tok_hbm, tbl_hbm, out_hbm, idx_v, rows_v, r128_v, sg0, sg1, so0, so1):
        wid = lax.axis_index("s") * NC + lax.axis_index("c")
        base = wid * RW
        semg = (sg0, sg1)
        semo = (so0, so1)

        pltpu.sync_copy(
            tok_hbm.at[pl.ds(pl.multiple_of(base // K, 8), RW // K)], idx_v)

        def fire(ci, b):
            for j in range(CH):
                pltpu.async_copy(
                    tbl_hbm.at[idx_v.at[ci * CH + j]],
                    rows_v.at[b, pl.ds(j * K, K)],
                    semg[b],
                )

        def wait_g(b):
            pltpu.make_async_copy(
                tbl_hbm.at[pl.ds(0, C)], rows_v.at[b], semg[b]).wait()

        def out_slice(ci):
            r0 = pl.multiple_of((base + ci * C) * EMB // 128, 8)
            return out_hbm.at[pl.ds(r0, C * EMB // 128)]

        def repack(b):
            def body(i, carry):
                r = i * 16
                vals = []
                for u in range(16):
                    for h in range(EMB // 16):
                        vals.append(rows_v[b, r + u, pl.ds(h * 16, 16)])
                for u in range(16):
                    for h in range(EMB // 16):
                        d = (b, r // 4 + u // 4,
                             pl.ds((u % 4) * EMB + h * 16, 16))
                        r128_v[d] = vals[u * (EMB // 16) + h]
                return carry
            lax.fori_loop(0, C // 16, body, 0)

        fire(0, 0)

        def outer(c2, carry):
            for b in range(2):
                ci = c2 * 2 + b
                nb = 1 - b

                @pl.when(ci + 1 < NCH)
                def _fire_next():
                    fire(ci + 1, nb)

                wait_g(b)

                @pl.when(ci >= 2)
                def _drain_out():
                    pltpu.make_async_copy(
                        r128_v.at[b], out_slice(ci - 2), semo[b]).wait()

                repack(b)
                pltpu.async_copy(r128_v.at[b], out_slice(ci), semo[b])
            return carry

        lax.fori_loop(0, NCH // 2, outer, 0)
        pltpu.make_async_copy(r128_v.at[0], out_slice(NCH - 2), semo[0]).wait()
        pltpu.make_async_copy(r128_v.at[1], out_slice(NCH - 1), semo[1]).wait()

    return k


def kernel(tokens, embedding):
    B = tokens.shape[0] * tokens.shape[1]
    V = embedding.shape[0]
    tok2d = tokens.reshape(B // K, K).astype(jnp.int32)
    tbl_scaled = _make_compactor(V)(embedding)
    tbl_lin = tbl_scaled.reshape(V, EMB)
    out128 = _make_gather(B, V)(tok2d, tbl_lin)
    return out128.reshape(tokens.shape[0], tokens.shape[1], EMB)
